# baseline (speedup 1.0000x reference)
import functools

import jax
import jax.numpy as jnp
from jax import lax
from jax.experimental import pallas as pl
from jax.experimental.pallas import tpu as pltpu
from jax.experimental.pallas import tpu_sc as plsc

N_SRC0 = 2500
N_DST = 512
E0 = 320000
E1 = 16384

NW = 32
NS = 16
CHUNK = 128

NCH0 = 80
PERW0 = NCH0 * CHUNK
E0P = NW * PERW0
NCH1 = 8
PERW1 = NCH1 * CHUNK
E1P = NW * PERW1

NA0 = N_DST * N_SRC0
NA1 = N_DST * N_DST
TRASH = NA0 + NA1
NTOT = NA0 + NA1 + 2560
ZSTRIPE = (NA0 + NA1) // NS
ZBUF = 8192
NZC = ZSTRIPE // ZBUF
ZTAIL = ZSTRIPE - NZC * ZBUF


@functools.partial(
    pl.kernel,
    out_type=(jax.ShapeDtypeStruct((2, NA0), jnp.float32),
              jax.ShapeDtypeStruct((2, NA1), jnp.float32)),
    mesh=plsc.VectorSubcoreMesh(core_axis_name="c", subcore_axis_name="s"),
    scratch_types=[
        pltpu.VMEM_SHARED((NTOT,), jnp.float32),
        pltpu.VMEM((NCH0, CHUNK), jnp.int32),
        pltpu.VMEM((NCH0, CHUNK), jnp.int32),
        pltpu.VMEM((NCH1, CHUNK), jnp.int32),
        pltpu.VMEM((NCH1, CHUNK), jnp.int32),
        pltpu.VMEM((CHUNK,), jnp.float32),
        pltpu.VMEM((ZBUF,), jnp.float32),
        pltpu.SemaphoreType.DMA,
        pltpu.SemaphoreType.DMA,
        pltpu.SemaphoreType.DMA,
    ],
)
def _sc_build(dst0, src0, dst1, src1, out0, out1, acc, dstv0, srcv0, dstv1,
              srcv1, ones, zeros, sem_st, sem_z, sem_sc):
    c = lax.axis_index("c")
    s = lax.axis_index("s")
    w = c * NS + s

    pltpu.async_copy(dst0.at[pl.ds(w * NCH0, NCH0)], dstv0, sem_st)
    pltpu.async_copy(src0.at[pl.ds(w * NCH0, NCH0)], srcv0, sem_st)
    pltpu.async_copy(dst1.at[pl.ds(w * NCH1, NCH1)], dstv1, sem_st)
    pltpu.async_copy(src1.at[pl.ds(w * NCH1, NCH1)], srcv1, sem_st)

    def fill_z(i, _):
        zeros[pl.ds(i * 16, 16)] = jnp.zeros((16,), jnp.float32)
        return 0
    lax.fori_loop(0, ZBUF // 16, fill_z, 0)
    for v in range(CHUNK // 16):
        ones[pl.ds(v * 16, 16)] = jnp.ones((16,), jnp.float32)

    def zclr(i, _):
        pltpu.async_copy(zeros, acc.at[pl.ds(s * ZSTRIPE + i * ZBUF, ZBUF)],
                         sem_z)
        return 0
    lax.fori_loop(0, NZC, zclr, 0)
    pltpu.async_copy(zeros.at[pl.ds(0, ZTAIL)],
                     acc.at[pl.ds(s * ZSTRIPE + NZC * ZBUF, ZTAIL)], sem_z)

    pltpu.make_async_copy(dst0.at[pl.ds(w * NCH0, NCH0)], dstv0, sem_st).wait()
    pltpu.make_async_copy(src0.at[pl.ds(w * NCH0, NCH0)], srcv0, sem_st).wait()
    pltpu.make_async_copy(dst1.at[pl.ds(w * NCH1, NCH1)], dstv1, sem_st).wait()
    pltpu.make_async_copy(src1.at[pl.ds(w * NCH1, NCH1)], srcv1, sem_st).wait()

    def body0(j, _):
        for v in range(CHUNK // 16):
            d = dstv0[j, pl.ds(v * 16, 16)]
            sv = srcv0[j, pl.ds(v * 16, 16)]
            flat = jnp.where(d < N_DST, d * N_SRC0 + sv, TRASH + sv)
            dstv0[j, pl.ds(v * 16, 16)] = flat
        return 0
    lax.fori_loop(0, NCH0, body0, 0)

    def body1(j, _):
        for v in range(CHUNK // 16):
            d = dstv1[j, pl.ds(v * 16, 16)]
            sv = srcv1[j, pl.ds(v * 16, 16)]
            dstv1[j, pl.ds(v * 16, 16)] = NA0 + d * N_DST + sv
        return 0
    lax.fori_loop(0, NCH1, body1, 0)

    def zdrain(i, _):
        pltpu.make_async_copy(
            zeros, acc.at[pl.ds(s * ZSTRIPE + i * ZBUF, ZBUF)], sem_z).wait()
        return 0
    lax.fori_loop(0, NZC, zdrain, 0)
    pltpu.make_async_copy(
        zeros.at[pl.ds(0, ZTAIL)],
        acc.at[pl.ds(s * ZSTRIPE + NZC * ZBUF, ZTAIL)], sem_z).wait()
    plsc.subcore_barrier()

    def fire0(j, _):
        pltpu.async_copy(ones, acc.at[dstv0.at[j]], sem_sc, add=True)
        return 0
    lax.fori_loop(0, NCH0, fire0, 0)

    def fire1(j, _):
        pltpu.async_copy(ones, acc.at[dstv1.at[j]], sem_sc, add=True)
        return 0
    lax.fori_loop(0, NCH1, fire1, 0)

    def drain(j, _):
        pltpu.make_async_copy(ones, acc.at[dstv0.at[0]], sem_sc).wait()
        return 0
    lax.fori_loop(0, NCH0 + NCH1, drain, 0)
    plsc.subcore_barrier()

    pltpu.async_copy(acc.at[pl.ds(s * (NA0 // NS), NA0 // NS)],
                     out0.at[c, pl.ds(s * (NA0 // NS), NA0 // NS)], sem_st)
    pltpu.async_copy(acc.at[pl.ds(NA0 + s * (NA1 // NS), NA1 // NS)],
                     out1.at[c, pl.ds(s * (NA1 // NS), NA1 // NS)], sem_st)
    pltpu.make_async_copy(acc.at[pl.ds(s * (NA0 // NS), NA0 // NS)],
                          out0.at[c, pl.ds(s * (NA0 // NS), NA0 // NS)],
                          sem_st).wait()
    pltpu.make_async_copy(acc.at[pl.ds(NA0 + s * (NA1 // NS), NA1 // NS)],
                          out1.at[c, pl.ds(s * (NA1 // NS), NA1 // NS)],
                          sem_st).wait()


def _tc_body(a0p, a1p, xr, wl0, wr0, b0r, wl1, wr1, b1r, out):
    f32 = jnp.float32
    hi = lax.Precision.HIGHEST
    x = xr[...]
    a0 = a0p[0] + a0p[1]
    cnt0 = jnp.maximum(jnp.sum(a0, axis=1, keepdims=True), 1.0)
    agg0 = jnp.dot(a0, x, precision=hi, preferred_element_type=f32) / cnt0
    h = (jnp.dot(agg0, wl0[...], precision=hi, preferred_element_type=f32)
         + b0r[...]
         + jnp.dot(x[:N_DST], wr0[...], precision=hi,
                   preferred_element_type=f32))
    h = jnp.maximum(h, 0.0)
    a1 = a1p[0] + a1p[1]
    cnt1 = jnp.maximum(jnp.sum(a1, axis=1, keepdims=True), 1.0)
    agg1 = jnp.dot(a1, h, precision=hi, preferred_element_type=f32) / cnt1
    o = (jnp.dot(agg1, wl1[...], precision=hi, preferred_element_type=f32)
         + b1r[...]
         + jnp.dot(h, wr1[...], precision=hi, preferred_element_type=f32))
    m = jnp.max(o, axis=1, keepdims=True)
    lse = jnp.log(jnp.sum(jnp.exp(o - m), axis=1, keepdims=True)) + m
    out[...] = o - lse


_tc = pl.pallas_call(
    _tc_body,
    out_shape=jax.ShapeDtypeStruct((N_DST, 128), jnp.float32),
)


@jax.jit
def kernel(x, edge_index0, edge_index1, Wl0, b0, Wr0, Wl1, b1, Wr1):
    ei0 = edge_index0.astype(jnp.int32)
    ei1 = edge_index1.astype(jnp.int32)
    spread0 = jnp.arange(E0P - E0, dtype=jnp.int32) % 2048
    spread1 = jnp.arange(E1P - E1, dtype=jnp.int32) % 2048
    dst0 = jnp.pad(ei0[1], (0, E0P - E0),
                   constant_values=N_DST).reshape(NW * NCH0, CHUNK)
    src0 = jnp.concatenate([ei0[0], spread0]).reshape(NW * NCH0, CHUNK)
    dst1 = jnp.pad(ei1[1], (0, E1P - E1),
                   constant_values=N_DST).reshape(NW * NCH1, CHUNK)
    src1 = jnp.concatenate([ei1[0], spread1]).reshape(NW * NCH1, CHUNK)
    a0f, a1f = _sc_build(dst0, src0, dst1, src1)
    a0p = a0f.reshape(2, N_DST, N_SRC0)
    a1p = a1f.reshape(2, N_DST, N_DST)
    return _tc(a0p, a1p, x[:N_SRC0], Wl0, Wr0, b0.reshape(1, -1),
               Wl1, Wr1, b1.reshape(1, -1))


# speedup vs baseline: 1.5318x; 1.5318x over previous
import functools

import jax
import jax.numpy as jnp
from jax import lax
from jax.experimental import pallas as pl
from jax.experimental.pallas import tpu as pltpu
from jax.experimental.pallas import tpu_sc as plsc

N_SRC0 = 2500
N_DST = 512
E0 = 320000
E1 = 16384

NS = 16
CHUNK = 128

NCH0 = 160
E0P = NS * NCH0 * CHUNK
NCH1 = 16
E1P = NS * NCH1 * CHUNK

HALF = N_DST // 2
RS0 = 2560
L1BASE = HALF * RS0
TRASH = L1BASE + HALF * N_DST
TRMASK = 2047
ACC = TRASH + TRMASK + 1
ZSTRIPE = TRASH // NS
ZBUF = 8192
NZC = ZSTRIPE // ZBUF
ROWS_T = HALF // NS


@functools.partial(
    pl.kernel,
    out_type=(jax.ShapeDtypeStruct((N_DST, RS0), jnp.float32),
              jax.ShapeDtypeStruct((N_DST, N_DST), jnp.float32)),
    mesh=plsc.VectorSubcoreMesh(core_axis_name="c", subcore_axis_name="s"),
    scratch_types=[
        pltpu.VMEM_SHARED((ACC,), jnp.float32),
        pltpu.VMEM((NCH0, CHUNK), jnp.int32),
        pltpu.VMEM((NCH0, CHUNK), jnp.int32),
        pltpu.VMEM((NCH1, CHUNK), jnp.int32),
        pltpu.VMEM((NCH1, CHUNK), jnp.int32),
        pltpu.VMEM((CHUNK,), jnp.float32),
        pltpu.VMEM((ZBUF,), jnp.float32),
        pltpu.VMEM((8, RS0), jnp.float32),
        pltpu.VMEM((8, N_DST), jnp.float32),
        pltpu.SemaphoreType.DMA,
        pltpu.SemaphoreType.DMA,
        pltpu.SemaphoreType.DMA,
    ],
)
def _sc_build(dst0, src0, dst1, src1, out0, out1, acc, dstv0, srcv0, dstv1,
              srcv1, ones, zeros, b0buf, b1buf, sem_st, sem_z, sem_sc):
    c = lax.axis_index("c")
    s = lax.axis_index("s")
    lo = c * HALF

    pltpu.async_copy(dst0.at[pl.ds(s * NCH0, NCH0)], dstv0, sem_st)
    pltpu.async_copy(src0.at[pl.ds(s * NCH0, NCH0)], srcv0, sem_st)
    pltpu.async_copy(dst1.at[pl.ds(s * NCH1, NCH1)], dstv1, sem_st)
    pltpu.async_copy(src1.at[pl.ds(s * NCH1, NCH1)], srcv1, sem_st)

    def fill_z(i, _):
        zeros[pl.ds(i * 16, 16)] = jnp.zeros((16,), jnp.float32)
        return 0
    lax.fori_loop(0, ZBUF // 16, fill_z, 0)
    for v in range(CHUNK // 16):
        ones[pl.ds(v * 16, 16)] = jnp.ones((16,), jnp.float32)

    def zclr(i, _):
        pltpu.async_copy(zeros, acc.at[pl.ds(s * ZSTRIPE + i * ZBUF, ZBUF)],
                         sem_z)
        return 0
    lax.fori_loop(0, NZC, zclr, 0)

    pltpu.make_async_copy(dst0.at[pl.ds(s * NCH0, NCH0)], dstv0, sem_st).wait()
    pltpu.make_async_copy(src0.at[pl.ds(s * NCH0, NCH0)], srcv0, sem_st).wait()
    pltpu.make_async_copy(dst1.at[pl.ds(s * NCH1, NCH1)], dstv1, sem_st).wait()
    pltpu.make_async_copy(src1.at[pl.ds(s * NCH1, NCH1)], srcv1, sem_st).wait()

    def body0(j, _):
        for v in range(CHUNK // 16):
            d = dstv0[j, pl.ds(v * 16, 16)]
            sv = srcv0[j, pl.ds(v * 16, 16)]
            mine = (d >= lo) & (d < lo + HALF)
            flat = jnp.where(mine, (d - lo) * RS0 + sv,
                             TRASH + (sv & TRMASK))
            dstv0[j, pl.ds(v * 16, 16)] = flat
        return 0
    lax.fori_loop(0, NCH0, body0, 0)

    def body1(j, _):
        for v in range(CHUNK // 16):
            d = dstv1[j, pl.ds(v * 16, 16)]
            sv = srcv1[j, pl.ds(v * 16, 16)]
            mine = (d >= lo) & (d < lo + HALF)
            flat = jnp.where(mine, L1BASE + (d - lo) * N_DST + sv,
                             TRASH + (sv & TRMASK))
            dstv1[j, pl.ds(v * 16, 16)] = flat
        return 0
    lax.fori_loop(0, NCH1, body1, 0)

    def zdrain(i, _):
        pltpu.make_async_copy(
            zeros, acc.at[pl.ds(s * ZSTRIPE + i * ZBUF, ZBUF)], sem_z).wait()
        return 0
    lax.fori_loop(0, NZC, zdrain, 0)
    plsc.subcore_barrier()

    def fire0(j, _):
        pltpu.async_copy(ones, acc.at[dstv0.at[j]], sem_sc, add=True)
        return 0
    lax.fori_loop(0, NCH0, fire0, 0)

    def fire1(j, _):
        pltpu.async_copy(ones, acc.at[dstv1.at[j]], sem_sc, add=True)
        return 0
    lax.fori_loop(0, NCH1, fire1, 0)

    def drain(j, _):
        pltpu.make_async_copy(ones, acc.at[dstv0.at[0]], sem_sc).wait()
        return 0
    lax.fori_loop(0, NCH0 + NCH1, drain, 0)
    plsc.subcore_barrier()

    for g in range(ROWS_T // 8):
        r0 = s * ROWS_T + g * 8
        for i in range(8):
            pltpu.async_copy(acc.at[pl.ds((r0 + i) * RS0, RS0)],
                             b0buf.at[i], sem_st)
            pltpu.async_copy(
                acc.at[pl.ds(L1BASE + (r0 + i) * N_DST, N_DST)],
                b1buf.at[i], sem_st)
        for i in range(8):
            pltpu.make_async_copy(acc.at[pl.ds((r0 + i) * RS0, RS0)],
                                  b0buf.at[i], sem_st).wait()
            pltpu.make_async_copy(
                acc.at[pl.ds(L1BASE + (r0 + i) * N_DST, N_DST)],
                b1buf.at[i], sem_st).wait()
        pltpu.sync_copy(b0buf, out0.at[pl.ds(lo + r0, 8), :])
        pltpu.sync_copy(b1buf, out1.at[pl.ds(lo + r0, 8), :])


def _tc_body(a0r, a1r, xr, wl0, wr0, b0r, wl1, wr1, b1r, out):
    f32 = jnp.float32
    hi = lax.Precision.HIGHEST
    x = xr[...]
    a0 = a0r[...]
    cnt0 = jnp.maximum(jnp.sum(a0, axis=1, keepdims=True), 1.0)
    agg0 = jnp.dot(a0, x, precision=hi, preferred_element_type=f32) / cnt0
    h = (jnp.dot(agg0, wl0[...], precision=hi, preferred_element_type=f32)
         + b0r[...]
         + jnp.dot(x[:N_DST], wr0[...], precision=hi,
                   preferred_element_type=f32))
    h = jnp.maximum(h, 0.0)
    a1 = a1r[...]
    cnt1 = jnp.maximum(jnp.sum(a1, axis=1, keepdims=True), 1.0)
    agg1 = jnp.dot(a1, h, precision=hi, preferred_element_type=f32) / cnt1
    o = (jnp.dot(agg1, wl1[...], precision=hi, preferred_element_type=f32)
         + b1r[...]
         + jnp.dot(h, wr1[...], precision=hi, preferred_element_type=f32))
    m = jnp.max(o, axis=1, keepdims=True)
    lse = jnp.log(jnp.sum(jnp.exp(o - m), axis=1, keepdims=True)) + m
    out[...] = o - lse


_tc = pl.pallas_call(
    _tc_body,
    grid=(1,),
    out_shape=jax.ShapeDtypeStruct((N_DST, 128), jnp.float32),
    in_specs=[
        pl.BlockSpec((N_DST, RS0), lambda i: (0, 0)),
        pl.BlockSpec((N_DST, N_DST), lambda i: (0, 0)),
        pl.BlockSpec((2560, 128), lambda i: (0, 0)),
        pl.BlockSpec((128, 128), lambda i: (0, 0)),
        pl.BlockSpec((128, 128), lambda i: (0, 0)),
        pl.BlockSpec((1, 128), lambda i: (0, 0)),
        pl.BlockSpec((128, 128), lambda i: (0, 0)),
        pl.BlockSpec((128, 128), lambda i: (0, 0)),
        pl.BlockSpec((1, 128), lambda i: (0, 0)),
    ],
    out_specs=pl.BlockSpec((N_DST, 128), lambda i: (0, 0)),
)


@jax.jit
def kernel(x, edge_index0, edge_index1, Wl0, b0, Wr0, Wl1, b1, Wr1):
    ei0 = edge_index0.astype(jnp.int32)
    ei1 = edge_index1.astype(jnp.int32)
    spread0 = jnp.arange(E0P - E0, dtype=jnp.int32) % (TRMASK + 1)
    spread1 = jnp.arange(E1P - E1, dtype=jnp.int32) % (TRMASK + 1)
    dst0 = jnp.pad(ei0[1], (0, E0P - E0),
                   constant_values=N_DST).reshape(NS * NCH0, CHUNK)
    src0 = jnp.concatenate([ei0[0], spread0]).reshape(NS * NCH0, CHUNK)
    dst1 = jnp.pad(ei1[1], (0, E1P - E1),
                   constant_values=N_DST).reshape(NS * NCH1, CHUNK)
    src1 = jnp.concatenate([ei1[0], spread1]).reshape(NS * NCH1, CHUNK)
    a0, a1 = _sc_build(dst0, src0, dst1, src1)
    return _tc(a0, a1, x, Wl0, Wr0, b0.reshape(1, -1),
               Wl1, Wr1, b1.reshape(1, -1))


# speedup vs baseline: 1.6513x; 1.0780x over previous
import functools

import jax
import jax.numpy as jnp
from jax import lax
from jax.experimental import pallas as pl
from jax.experimental.pallas import tpu as pltpu
from jax.experimental.pallas import tpu_sc as plsc

N_SRC0 = 2500
N_DST = 512
E0 = 320000
E1 = 16384

NS = 16
CHUNK = 128

PCH0 = 80
SPAN0 = PCH0 * CHUNK
E0P = NS * 2 * SPAN0
NCH1 = 16
SPAN1 = NCH1 * CHUNK
E1P = NS * SPAN1

HALF = N_DST // 2
RS0 = 2560
L1BASE = HALF * RS0
TRASH = L1BASE + HALF * N_DST
TRMASK = 2047
ACC = TRASH + TRMASK + 1
ZSTRIPE = TRASH // NS
ZBUF = 2048
NZC = ZSTRIPE // ZBUF
ROWS_T = HALF // NS


@functools.partial(
    pl.kernel,
    out_type=(jax.ShapeDtypeStruct((N_DST, RS0), jnp.float32),
              jax.ShapeDtypeStruct((N_DST, N_DST), jnp.float32)),
    mesh=plsc.VectorSubcoreMesh(core_axis_name="c", subcore_axis_name="s"),
    scratch_types=[
        pltpu.VMEM_SHARED((ACC,), jnp.float32),
        pltpu.VMEM((2, SPAN0), jnp.int32),
        pltpu.VMEM((PCH0, CHUNK), jnp.int32),
        pltpu.VMEM((PCH0, CHUNK), jnp.int32),
        pltpu.VMEM((2, SPAN1), jnp.int32),
        pltpu.VMEM((NCH1, CHUNK), jnp.int32),
        pltpu.VMEM((CHUNK,), jnp.float32),
        pltpu.VMEM((ZBUF,), jnp.float32),
        pltpu.VMEM((8, RS0), jnp.float32),
        pltpu.VMEM((8, N_DST), jnp.float32),
        pltpu.SemaphoreType.DMA,
        pltpu.SemaphoreType.DMA,
        pltpu.SemaphoreType.DMA,
    ],
)
def _sc_build(e0, e1, out0, out1, acc, ev0, idx0a, idx0b, ev1, idx1,
              ones, zeros, b0buf, b1buf, sem_st, sem_z, sem_sc):
    c = lax.axis_index("c")
    s = lax.axis_index("s")
    lo = c * HALF
    base0 = s * 2 * SPAN0

    pltpu.async_copy(e0.at[:, pl.ds(base0, SPAN0)], ev0, sem_st)
    pltpu.async_copy(e1.at[:, pl.ds(s * SPAN1, SPAN1)], ev1, sem_st)

    def fill_z(i, _):
        zeros[pl.ds(i * 16, 16)] = jnp.zeros((16,), jnp.float32)
        return 0
    lax.fori_loop(0, ZBUF // 16, fill_z, 0)
    for v in range(CHUNK // 16):
        ones[pl.ds(v * 16, 16)] = jnp.ones((16,), jnp.float32)

    def zclr(i, _):
        pltpu.async_copy(zeros, acc.at[pl.ds(s * ZSTRIPE + i * ZBUF, ZBUF)],
                         sem_z)
        return 0
    lax.fori_loop(0, NZC, zclr, 0)

    pltpu.make_async_copy(e0.at[:, pl.ds(base0, SPAN0)], ev0, sem_st).wait()
    pltpu.make_async_copy(e1.at[:, pl.ds(s * SPAN1, SPAN1)], ev1,
                          sem_st).wait()

    def mkidx0(idx):
        def body(j, _):
            for v in range(CHUNK // 16):
                d = ev0[1, pl.ds(j * CHUNK + v * 16, 16)]
                sv = ev0[0, pl.ds(j * CHUNK + v * 16, 16)]
                mine = (d >= lo) & (d < lo + HALF)
                idx[j, pl.ds(v * 16, 16)] = jnp.where(
                    mine, (d - lo) * RS0 + sv, TRASH + (sv & TRMASK))
            return 0
        lax.fori_loop(0, PCH0, body, 0)

    mkidx0(idx0a)

    def body1(j, _):
        for v in range(CHUNK // 16):
            d = ev1[1, pl.ds(j * CHUNK + v * 16, 16)]
            sv = ev1[0, pl.ds(j * CHUNK + v * 16, 16)]
            mine = (d >= lo) & (d < lo + HALF)
            idx1[j, pl.ds(v * 16, 16)] = jnp.where(
                mine, L1BASE + (d - lo) * N_DST + sv, TRASH + (sv & TRMASK))
        return 0
    lax.fori_loop(0, NCH1, body1, 0)

    pltpu.async_copy(e0.at[:, pl.ds(base0 + SPAN0, SPAN0)], ev0, sem_st)

    def zdrain(i, _):
        pltpu.make_async_copy(
            zeros, acc.at[pl.ds(s * ZSTRIPE + i * ZBUF, ZBUF)], sem_z).wait()
        return 0
    lax.fori_loop(0, NZC, zdrain, 0)
    plsc.subcore_barrier()

    def fire0a(j, _):
        pltpu.async_copy(ones, acc.at[idx0a.at[j]], sem_sc, add=True)
        return 0
    lax.fori_loop(0, PCH0, fire0a, 0)

    def fire1(j, _):
        pltpu.async_copy(ones, acc.at[idx1.at[j]], sem_sc, add=True)
        return 0
    lax.fori_loop(0, NCH1, fire1, 0)

    pltpu.make_async_copy(e0.at[:, pl.ds(base0 + SPAN0, SPAN0)], ev0,
                          sem_st).wait()
    mkidx0(idx0b)

    def fire0b(j, _):
        pltpu.async_copy(ones, acc.at[idx0b.at[j]], sem_sc, add=True)
        return 0
    lax.fori_loop(0, PCH0, fire0b, 0)

    def drain(j, _):
        pltpu.make_async_copy(ones, acc.at[idx0a.at[0]], sem_sc).wait()
        return 0
    lax.fori_loop(0, 2 * PCH0 + NCH1, drain, 0)
    plsc.subcore_barrier()

    for g in range(ROWS_T // 8):
        r0 = s * ROWS_T + g * 8
        for i in range(8):
            pltpu.async_copy(acc.at[pl.ds((r0 + i) * RS0, RS0)],
                             b0buf.at[i], sem_st)
            pltpu.async_copy(
                acc.at[pl.ds(L1BASE + (r0 + i) * N_DST, N_DST)],
                b1buf.at[i], sem_st)
        for i in range(8):
            pltpu.make_async_copy(acc.at[pl.ds((r0 + i) * RS0, RS0)],
                                  b0buf.at[i], sem_st).wait()
            pltpu.make_async_copy(
                acc.at[pl.ds(L1BASE + (r0 + i) * N_DST, N_DST)],
                b1buf.at[i], sem_st).wait()
        pltpu.sync_copy(b0buf, out0.at[pl.ds(lo + r0, 8), :])
        pltpu.sync_copy(b1buf, out1.at[pl.ds(lo + r0, 8), :])


def _tc_body(a0r, a1r, xr, wl0, wr0, b0r, wl1, wr1, b1r, out):
    f32 = jnp.float32
    hi = lax.Precision.HIGHEST
    x = xr[...]
    a0 = a0r[...]
    cnt0 = jnp.maximum(jnp.sum(a0, axis=1, keepdims=True), 1.0)
    agg0 = jnp.dot(a0, x, precision=hi, preferred_element_type=f32) / cnt0
    h = (jnp.dot(agg0, wl0[...], precision=hi, preferred_element_type=f32)
         + b0r[...]
         + jnp.dot(x[:N_DST], wr0[...], precision=hi,
                   preferred_element_type=f32))
    h = jnp.maximum(h, 0.0)
    a1 = a1r[...]
    cnt1 = jnp.maximum(jnp.sum(a1, axis=1, keepdims=True), 1.0)
    agg1 = jnp.dot(a1, h, precision=hi, preferred_element_type=f32) / cnt1
    o = (jnp.dot(agg1, wl1[...], precision=hi, preferred_element_type=f32)
         + b1r[...]
         + jnp.dot(h, wr1[...], precision=hi, preferred_element_type=f32))
    m = jnp.max(o, axis=1, keepdims=True)
    lse = jnp.log(jnp.sum(jnp.exp(o - m), axis=1, keepdims=True)) + m
    out[...] = o - lse


_tc = pl.pallas_call(
    _tc_body,
    grid=(1,),
    out_shape=jax.ShapeDtypeStruct((N_DST, 128), jnp.float32),
    in_specs=[
        pl.BlockSpec((N_DST, RS0), lambda i: (0, 0)),
        pl.BlockSpec((N_DST, N_DST), lambda i: (0, 0)),
        pl.BlockSpec((RS0, 128), lambda i: (0, 0)),
        pl.BlockSpec((128, 128), lambda i: (0, 0)),
        pl.BlockSpec((128, 128), lambda i: (0, 0)),
        pl.BlockSpec((1, 128), lambda i: (0, 0)),
        pl.BlockSpec((128, 128), lambda i: (0, 0)),
        pl.BlockSpec((128, 128), lambda i: (0, 0)),
        pl.BlockSpec((1, 128), lambda i: (0, 0)),
    ],
    out_specs=pl.BlockSpec((N_DST, 128), lambda i: (0, 0)),
)


@jax.jit
def kernel(x, edge_index0, edge_index1, Wl0, b0, Wr0, Wl1, b1, Wr1):
    ei0 = edge_index0.astype(jnp.int32)
    ei1 = edge_index1.astype(jnp.int32)
    pad0 = jnp.stack([jnp.arange(E0P - E0, dtype=jnp.int32) & TRMASK,
                      jnp.full((E0P - E0,), N_DST, jnp.int32)])
    pad1 = jnp.stack([jnp.arange(E1P - E1, dtype=jnp.int32) & TRMASK,
                      jnp.full((E1P - E1,), N_DST, jnp.int32)])
    e0 = jnp.concatenate([ei0, pad0], axis=1)
    e1 = jnp.concatenate([ei1, pad1], axis=1)
    a0, a1 = _sc_build(e0, e1)
    return _tc(a0, a1, x, Wl0, Wr0, b0.reshape(1, -1),
               Wl1, Wr1, b1.reshape(1, -1))
